# baseline (device time: 45323 ns/iter reference)
import jax
import jax.numpy as jnp
from jax import lax
from jax.experimental import pallas as pl
from jax.experimental.pallas import tpu as pltpu

NH = 8


def kernel(Q, K, V):
    b, s, h, d = Q.shape
    scale = d ** -0.5

    def body(q_ref, k_ref, v_ref, out_ref,
             q_loc, k_loc, v_loc, o_mine, o_theirs,
             kv_send, kv_recv, o_send, o_recv,
             q_dsem, k_dsem, v_dsem, om_dsem, ot_dsem,
             kv_ssem, kv_rsem, o_ssem, o_rsem):
        my_x = lax.axis_index("x")
        my_y = lax.axis_index("y")
        y_nbr = (my_x, 1 - my_y)
        x_nbr = (1 - my_x, my_y)
        bm = my_x
        bo = 1 - my_x

        k_dmas, v_dmas, q_dmas = [], [], []
        for c in range(NH):
            r = pltpu.make_async_copy(
                k_ref.at[bm, :, c, :], k_loc.at[c], k_dsem.at[c])
            r.start()
            k_dmas.append(r)
            r = pltpu.make_async_copy(
                v_ref.at[bm, :, c, :], v_loc.at[c], v_dsem.at[c])
            r.start()
            v_dmas.append(r)
            r = pltpu.make_async_copy(
                q_ref.at[bm, :, c, :], q_loc.at[c], q_dsem.at[c])
            r.start()
            q_dmas.append(r)

        barrier = pltpu.get_barrier_semaphore()
        for nbr in (y_nbr, x_nbr):
            pl.semaphore_signal(
                barrier, inc=1, device_id=nbr,
                device_id_type=pl.DeviceIdType.MESH,
            )
        pl.semaphore_wait(barrier, 2)

        kv_rdmas = []
        for c in range(NH):
            k_dmas[c].wait()
            v_dmas[c].wait()
            kv_send[c, 0] = k_loc[c].astype(jnp.bfloat16)
            kv_send[c, 1] = v_loc[c].astype(jnp.bfloat16)
            r = pltpu.make_async_remote_copy(
                src_ref=kv_send.at[c], dst_ref=kv_recv.at[c],
                send_sem=kv_ssem.at[c], recv_sem=kv_rsem.at[c],
                device_id=y_nbr, device_id_type=pl.DeviceIdType.MESH,
            )
            r.start()
            kv_rdmas.append(r)

        o_rdmas = []
        om_dmas = []
        ot_dmas = [None] * NH
        for c in range(NH):
            kv_rdmas[c].wait_recv()
            q_dmas[c].wait()
            k_all = jnp.concatenate([kv_send[c, 0], kv_recv[c, 0]], axis=0)
            v_all = jnp.concatenate([kv_send[c, 1], kv_recv[c, 1]], axis=0)
            s_i = lax.dot_general(
                q_loc[c].astype(jnp.bfloat16), k_all,
                (((1,), (1,)), ((), ())),
                preferred_element_type=jnp.float32,
            ) * scale
            p = jnp.exp(s_i)
            l = jnp.sum(p, axis=1, keepdims=True)
            o_i = lax.dot_general(
                p.astype(jnp.bfloat16), v_all, (((1,), (0,)), ((), ())),
                preferred_element_type=jnp.float32,
            ) / l
            o_mine[c] = o_i
            r = pltpu.make_async_copy(
                o_mine.at[c], out_ref.at[bm, :, c, :], om_dsem.at[c])
            r.start()
            om_dmas.append(r)
            o_send[c] = o_i.astype(jnp.bfloat16)
            r = pltpu.make_async_remote_copy(
                src_ref=o_send.at[c], dst_ref=o_recv.at[c],
                send_sem=o_ssem.at[c], recv_sem=o_rsem.at[c],
                device_id=x_nbr, device_id_type=pl.DeviceIdType.MESH,
            )
            r.start()
            o_rdmas.append(r)
            if c > 0:
                o_rdmas[c - 1].wait_recv()
                o_theirs[c - 1] = o_recv[c - 1].astype(jnp.float32)
                r = pltpu.make_async_copy(
                    o_theirs.at[c - 1], out_ref.at[bo, :, c - 1, :],
                    ot_dsem.at[c - 1])
                r.start()
                ot_dmas[c - 1] = r

        o_rdmas[NH - 1].wait_recv()
        o_theirs[NH - 1] = o_recv[NH - 1].astype(jnp.float32)
        r = pltpu.make_async_copy(
            o_theirs.at[NH - 1], out_ref.at[bo, :, NH - 1, :],
            ot_dsem.at[NH - 1])
        r.start()
        ot_dmas[NH - 1] = r

        for c in range(NH):
            kv_rdmas[c].wait_send()
            o_rdmas[c].wait_send()
            om_dmas[c].wait()
            ot_dmas[c].wait()

    return pl.pallas_call(
        body,
        out_shape=jax.ShapeDtypeStruct((b, s, h, d), jnp.float32),
        in_specs=[pl.BlockSpec(memory_space=pl.ANY)] * 3,
        out_specs=pl.BlockSpec(memory_space=pl.ANY),
        scratch_shapes=[
            pltpu.VMEM((NH, s, d), jnp.float32),
            pltpu.VMEM((NH, s, d), jnp.float32),
            pltpu.VMEM((NH, s, d), jnp.float32),
            pltpu.VMEM((NH, s, d), jnp.float32),
            pltpu.VMEM((NH, s, d), jnp.float32),
            pltpu.VMEM((NH, 2, s, d), jnp.bfloat16),
            pltpu.VMEM((NH, 2, s, d), jnp.bfloat16),
            pltpu.VMEM((NH, s, d), jnp.bfloat16),
            pltpu.VMEM((NH, s, d), jnp.bfloat16),
            pltpu.SemaphoreType.DMA((NH,)),
            pltpu.SemaphoreType.DMA((NH,)),
            pltpu.SemaphoreType.DMA((NH,)),
            pltpu.SemaphoreType.DMA((NH,)),
            pltpu.SemaphoreType.DMA((NH,)),
            pltpu.SemaphoreType.DMA((NH,)),
            pltpu.SemaphoreType.DMA((NH,)),
            pltpu.SemaphoreType.DMA((NH,)),
            pltpu.SemaphoreType.DMA((NH,)),
        ],
        compiler_params=pltpu.CompilerParams(collective_id=0),
    )(Q, K, V)


# device time: 40328 ns/iter; 1.1239x vs baseline; 1.1239x over previous
import jax
import jax.numpy as jnp
from jax import lax
from jax.experimental import pallas as pl
from jax.experimental.pallas import tpu as pltpu

HALF_ROWS = 8
NCHUNK = 8


def kernel(Q, K, V):
    b, s, h, d = Q.shape
    hb = h * b
    scale = d ** -0.5

    def to_rows(A):
        return jnp.transpose(A.astype(jnp.bfloat16), (2, 0, 1, 3)).reshape(
            hb, s, d)

    def body(q_ref, k_ref, v_ref, out_ref, kv_recv, o_recv,
             kv_ssem, kv_rsem, o_ssem, o_rsem):
        my_x = lax.axis_index("x")
        my_y = lax.axis_index("y")
        y_nbr = (my_x, 1 - my_y)
        x_nbr = (1 - my_x, my_y)

        barrier = pltpu.get_barrier_semaphore()
        for nbr in (y_nbr, x_nbr):
            pl.semaphore_signal(
                barrier, inc=1, device_id=nbr,
                device_id_type=pl.DeviceIdType.MESH,
            )
        pl.semaphore_wait(barrier, 2)

        base = my_x * HALF_ROWS
        other = (1 - my_x) * HALF_ROWS

        kv_rdmas = []
        for c in range(NCHUNK):
            r = pltpu.make_async_remote_copy(
                src_ref=k_ref.at[pl.ds(base + c, 1)],
                dst_ref=kv_recv.at[c, 0:1],
                send_sem=kv_ssem.at[2 * c], recv_sem=kv_rsem.at[2 * c],
                device_id=y_nbr, device_id_type=pl.DeviceIdType.MESH,
            )
            r.start()
            kv_rdmas.append(r)
            r = pltpu.make_async_remote_copy(
                src_ref=v_ref.at[pl.ds(base + c, 1)],
                dst_ref=kv_recv.at[c, 1:2],
                send_sem=kv_ssem.at[2 * c + 1], recv_sem=kv_rsem.at[2 * c + 1],
                device_id=y_nbr, device_id_type=pl.DeviceIdType.MESH,
            )
            r.start()
            kv_rdmas.append(r)

        qh = q_ref[pl.ds(base, HALF_ROWS)]
        kh = k_ref[pl.ds(base, HALF_ROWS)]
        vh = v_ref[pl.ds(base, HALF_ROWS)]

        o_rdmas = []
        for c in range(NCHUNK):
            kv_rdmas[2 * c].wait_recv()
            kv_rdmas[2 * c + 1].wait_recv()
            k_all = jnp.concatenate([kh[c], kv_recv[c, 0]], axis=0)
            v_all = jnp.concatenate([vh[c], kv_recv[c, 1]], axis=0)
            s_i = lax.dot_general(
                qh[c], k_all, (((1,), (1,)), ((), ())),
                preferred_element_type=jnp.float32,
            ) * scale
            p = jnp.exp(s_i)
            l = jnp.sum(p, axis=1, keepdims=True)
            o_i = lax.dot_general(
                p.astype(jnp.bfloat16), v_all, (((1,), (0,)), ((), ())),
                preferred_element_type=jnp.float32,
            ) / l
            out_ref[pl.ds(base + c, 1)] = o_i.astype(jnp.bfloat16)[None]
            r = pltpu.make_async_remote_copy(
                src_ref=out_ref.at[pl.ds(base + c, 1)],
                dst_ref=o_recv.at[c, 0:1],
                send_sem=o_ssem.at[c], recv_sem=o_rsem.at[c],
                device_id=x_nbr, device_id_type=pl.DeviceIdType.MESH,
            )
            r.start()
            o_rdmas.append(r)
            if c > 0:
                o_rdmas[c - 1].wait_recv()
                out_ref[pl.ds(other + c - 1, 1)] = o_recv[c - 1]

        o_rdmas[NCHUNK - 1].wait_recv()
        out_ref[pl.ds(other + NCHUNK - 1, 1)] = o_recv[NCHUNK - 1]

        for c in range(NCHUNK):
            kv_rdmas[2 * c].wait_send()
            kv_rdmas[2 * c + 1].wait_send()
            o_rdmas[c].wait_send()

    out_rows = pl.pallas_call(
        body,
        out_shape=jax.ShapeDtypeStruct((hb, s, d), jnp.bfloat16),
        in_specs=[pl.BlockSpec(memory_space=pltpu.VMEM)] * 3,
        out_specs=pl.BlockSpec(memory_space=pltpu.VMEM),
        scratch_shapes=[
            pltpu.VMEM((NCHUNK, 2, s, d), jnp.bfloat16),
            pltpu.VMEM((NCHUNK, 1, s, d), jnp.bfloat16),
            pltpu.SemaphoreType.DMA((2 * NCHUNK,)),
            pltpu.SemaphoreType.DMA((2 * NCHUNK,)),
            pltpu.SemaphoreType.DMA((NCHUNK,)),
            pltpu.SemaphoreType.DMA((NCHUNK,)),
        ],
        compiler_params=pltpu.CompilerParams(collective_id=0),
    )(to_rows(Q), to_rows(K), to_rows(V))

    return jnp.transpose(out_rows.reshape(h, b, s, d), (1, 2, 0, 3)).astype(
        jnp.float32)
